# baseline (speedup 1.0000x reference)
import jax
import jax.numpy as jnp
from jax import lax
from jax.experimental import pallas as pl
from jax.experimental.pallas import tpu as pltpu
from jax.experimental.pallas import tpu_sc as plsc

_BIG = jnp.int32(0x7FFFFFFF)


def _make_topk(B, N, K):
    info = plsc.get_sparse_core_info()
    NC, NS = info.num_cores, info.num_subcores
    mesh = plsc.VectorSubcoreMesh(core_axis_name="c", subcore_axis_name="s")
    NV = N // 16
    KV = K // 16

    def body(xs_hbm, gidx_hbm, xso_hbm, kd_ref, bb_ref, h16_ref, ck_ref,
             ci_ref, ak_ref, ai_ref, hist_ref, sbuf_ref):
        wid = lax.axis_index("s") * NC + lax.axis_index("c")
        lane = lax.iota(jnp.int32, 16)

        def _dedup(d, valid):
            if valid is None:
                ukey = d * 16 + lane
            else:
                ukey = jnp.where(valid, d * 16 + lane, _BIG)
            (sk,) = lax.sort([ukey], dimension=0, num_keys=1)
            sbuf_ref[pl.ds(0, 16)] = jnp.full((16,), -1, jnp.int32)
            sbuf_ref[pl.ds(1, 16)] = sk
            prev = plsc.load_gather(sbuf_ref, [lane])
            nxt = plsc.load_gather(sbuf_ref, [jnp.minimum(lane + 2, 16)])
            sd = sk >> 4
            boundary = sd != (prev >> 4)
            fpos = plsc.cummax(jnp.where(boundary, lane, 0))
            rk_sorted = lane - fpos
            lastm = (lane == 15) | ((nxt >> 4) != sd)
            if valid is not None:
                lastm = lastm & (sk != _BIG)
            return sk, sd, lastm, rk_sorted

        def _hist_add(d, valid):
            _, sd, lastm, rk_sorted = _dedup(d, valid)
            h = plsc.load_gather(hist_ref, [sd], mask=lastm)
            plsc.store_scatter(hist_ref, [sd], h + rk_sorted + 1, mask=lastm)

        def _zero_hist(nb):
            def z(i, c):
                hist_ref[pl.ds(i * 16, 16)] = jnp.zeros((16,), jnp.int32)
                return c
            lax.fori_loop(0, nb // 16, z, 0)

        def _scan_hist(nb, t):
            def sb(i, carry):
                run, nf, g = carry
                h = hist_ref[pl.ds(i * 16, 16)]
                c = plsc.cumsum(h) + run
                le = c <= t
                nf = nf + jnp.sum(jnp.where(le, 1, 0).astype(jnp.int32))
                g = jnp.maximum(g, jnp.max(jnp.where(le, c, 0)))
                return jnp.max(c), nf, g
            z = jnp.int32(0)
            _, bstar, g = lax.fori_loop(0, nb // 16, sb, (z, z, z))
            return bstar, g

        @pl.when(wid < B)
        def _():
            r = wid
            pltpu.sync_copy(xs_hbm.at[pl.ds(r * N, N)], kd_ref)

            def z1(i, c):
                for u in range(8):
                    h16_ref[pl.ds(i * 128 + u * 16, 16)] = (
                        jnp.zeros((16,), jnp.int32))
                return c
            lax.fori_loop(0, 256, z1, 0)

            def p1(i, c):
                for u in range(4):
                    uu = kd_ref[pl.ds(i * 64 + u * 16, 16)]
                    kd = ~(uu ^ ((uu >> 31) & _BIG))
                    kd_ref[pl.ds(i * 64 + u * 16, 16)] = kd
                    a = lane * 2048 + ((kd >> 21) + 1024)
                    g = plsc.load_gather(h16_ref, [a])
                    plsc.store_scatter(h16_ref, [a], g + 1)
                return c
            lax.fori_loop(0, NV // 4, p1, 0)

            t1 = jnp.int32(K - 1)

            def s1(j, carry):
                run, nf, g = carry
                tot = h16_ref[pl.ds(j * 16, 16)]
                for l in range(1, 16):
                    tot = tot + h16_ref[pl.ds(l * 2048 + j * 16, 16)]
                c = plsc.cumsum(tot) + run
                le = c <= t1
                nf = nf + jnp.sum(jnp.where(le, 1, 0).astype(jnp.int32))
                g = jnp.maximum(g, jnp.max(jnp.where(le, c, 0)))
                return jnp.max(c), nf, g
            z = jnp.int32(0)
            _, b1, g1 = lax.fori_loop(0, 128, s1, (z, z, z))
            b1v = b1 - 1024
            t2 = t1 - g1

            def cbb(i, off):
                for u in range(4):
                    kd = kd_ref[pl.ds(i * 64 + u * 16, 16)]
                    m = (kd >> 21) == b1v
                    plsc.store_compressed(bb_ref.at[pl.ds(off, 16)], kd,
                                          mask=m)
                    off = off + jnp.sum(jnp.where(m, 1, 0).astype(jnp.int32))
                return off
            m1 = lax.fori_loop(0, NV // 4, cbb, jnp.int32(0))
            m1v = (m1 + 15) // 16

            _zero_hist(2048)

            def h2(i, c):
                kd = bb_ref[pl.ds(i * 16, 16)]
                _hist_add((kd >> 10) & 0x7FF, (i * 16 + lane) < m1)
                return c
            lax.fori_loop(0, m1v, h2, 0)
            b2, g2 = _scan_hist(2048, t2)
            t3 = t2 - g2

            def c2(i, off):
                kd = bb_ref[pl.ds(i * 16, 16)]
                m = ((i * 16 + lane) < m1) & (((kd >> 10) & 0x7FF) == b2)
                plsc.store_compressed(bb_ref.at[pl.ds(off, 16)], kd, mask=m)
                return off + jnp.sum(jnp.where(m, 1, 0).astype(jnp.int32))
            m2 = lax.fori_loop(0, m1v, c2, jnp.int32(0))
            m2v = (m2 + 15) // 16

            _zero_hist(1024)

            def h3(i, c):
                kd = bb_ref[pl.ds(i * 16, 16)]
                _hist_add(kd & 0x3FF, (i * 16 + lane) < m2)
                return c
            lax.fori_loop(0, m2v, h3, 0)
            b3, g3 = _scan_hist(1024, t3)

            T = (b1v << 21) | (b2 << 10) | b3
            need = jnp.int32(K) - (g1 + g2 + g3)

            def cb(i, carry):
                off, trun = carry
                for u in range(4):
                    kd = kd_ref[pl.ds(i * 64 + u * 16, 16)]
                    strict = kd < T
                    tie = kd == T
                    tord = plsc.cumsum(jnp.where(tie, 1, 0).astype(jnp.int32))
                    msel = strict | (tie & ((tord + trun) <= need))
                    gi = i * 64 + u * 16 + lane
                    plsc.store_compressed(ck_ref.at[pl.ds(off, 16)], kd,
                                          mask=msel)
                    plsc.store_compressed(ci_ref.at[pl.ds(off, 16)], gi,
                                          mask=msel)
                    off = off + jnp.sum(jnp.where(msel, 1, 0).astype(jnp.int32))
                    trun = trun + jnp.max(tord)
                return off, trun
            lax.fori_loop(0, NV // 4, cb, (jnp.int32(0), jnp.int32(0)))

            for p in range(4):
                src_k, src_i = (ck_ref, ci_ref) if p % 2 == 0 else (ak_ref, ai_ref)
                dst_k, dst_i = (ak_ref, ai_ref) if p % 2 == 0 else (ck_ref, ci_ref)
                sh = 8 * p

                def dig(kd, sh=sh, p=p):
                    d = (kd >> sh) & 255
                    if p == 3:
                        d = d ^ 128
                    return d

                def zr(i, c):
                    for u in range(8):
                        h16_ref[pl.ds(i * 128 + u * 16, 16)] = (
                            jnp.zeros((16,), jnp.int32))
                    return c
                lax.fori_loop(0, 128, zr, 0)

                q = KV // 4

                def hb(i, c, src_k=src_k, dig=dig):
                    for u in range(4):
                        kd = plsc.load_gather(src_k, [lane * KV + u * q + i])
                        a = u * 4096 + dig(kd) * 16 + lane
                        g = plsc.load_gather(h16_ref, [a])
                        plsc.store_scatter(h16_ref, [a], g + 1)
                    return c
                lax.fori_loop(0, q, hb, 0)

                def pb(d, run):
                    v0 = h16_ref[pl.ds(d * 16, 16)]
                    v1 = h16_ref[pl.ds(4096 + d * 16, 16)]
                    v2 = h16_ref[pl.ds(8192 + d * 16, 16)]
                    v3 = h16_ref[pl.ds(12288 + d * 16, 16)]
                    tot = v0 + v1 + v2 + v3
                    acc = plsc.cumsum(tot) - tot + run
                    bb_ref[pl.ds(d * 16, 16)] = acc
                    acc = acc + v0
                    bb_ref[pl.ds(4096 + d * 16, 16)] = acc
                    acc = acc + v1
                    bb_ref[pl.ds(8192 + d * 16, 16)] = acc
                    acc = acc + v2
                    bb_ref[pl.ds(12288 + d * 16, 16)] = acc
                    return run + jnp.sum(tot)
                lax.fori_loop(0, 256, pb, jnp.int32(0))

                def mb(i, c, src_k=src_k, src_i=src_i, dst_k=dst_k,
                       dst_i=dst_i, dig=dig):
                    for u in range(4):
                        kd = plsc.load_gather(src_k, [lane * KV + u * q + i])
                        ix = plsc.load_gather(src_i, [lane * KV + u * q + i])
                        a = u * 4096 + dig(kd) * 16 + lane
                        pos = plsc.load_gather(bb_ref, [a])
                        plsc.store_scatter(bb_ref, [a], pos + 1)
                        plsc.store_scatter(dst_k, [pos], kd)
                        plsc.store_scatter(dst_i, [pos], ix)
                    return c
                lax.fori_loop(0, q, mb, 0)

            def ob(i, c):
                for u in range(4):
                    kd = ck_ref[pl.ds(i * 64 + u * 16, 16)]
                    s = ~kd
                    ck_ref[pl.ds(i * 64 + u * 16, 16)] = s ^ ((s >> 31) & _BIG)
                return c
            lax.fori_loop(0, KV // 4, ob, 0)
            pltpu.sync_copy(ck_ref.at[pl.ds(0, K)], xso_hbm.at[pl.ds(r * K, K)])
            pltpu.sync_copy(ci_ref.at[pl.ds(0, K)], gidx_hbm.at[pl.ds(r * K, K)])

    return pl.kernel(
        body,
        out_type=(
            jax.ShapeDtypeStruct((B * K,), jnp.int32),
            jax.ShapeDtypeStruct((B * K,), jnp.int32),
        ),
        mesh=mesh,
        scratch_types=[
            pltpu.VMEM((N,), jnp.int32),
            pltpu.VMEM((N,), jnp.int32),
            pltpu.VMEM((16 * 2048,), jnp.int32),
            pltpu.VMEM((K + 16,), jnp.int32),
            pltpu.VMEM((K + 16,), jnp.int32),
            pltpu.VMEM((K + 16,), jnp.int32),
            pltpu.VMEM((K + 16,), jnp.int32),
            pltpu.VMEM((2048,), jnp.int32),
            pltpu.VMEM((48,), jnp.int32),
        ],
        compiler_params=pltpu.CompilerParams(needs_layout_passes=False),
    )


def _make_gather(B, N, K, D, Dv):
    info = plsc.get_sparse_core_info()
    NC, NS = info.num_cores, info.num_subcores
    mesh = plsc.VectorSubcoreMesh(core_axis_name="c", subcore_axis_name="s")
    KV = K // 16

    def body(xT_hbm, xvT_hbm, gidx_hbm, xoT_hbm, xvoT_hbm, idx_ref, rb0, rb1,
             rb2, ob, sem0, sem1, sem2):
        wid = lax.axis_index("s") * NC + lax.axis_index("c")
        b = wid // 2
        h = wid % 2

        pltpu.sync_copy(gidx_hbm.at[pl.ds(b * K, K)], idx_ref)

        def gather_row(rbuf):
            def g(j, c):
                for u in range(4):
                    iv = idx_ref[pl.ds(j * 64 + u * 16, 16)]
                    ob[pl.ds(j * 64 + u * 16, 16)] = (
                        plsc.load_gather(rbuf, [iv]))
                return c
            lax.fori_loop(0, KV // 4, g, 0)

        rbs = (rb0, rb1, rb2)
        sems = (sem0, sem1, sem2)

        def phase(src, dst, row0, nrows):
            pltpu.async_copy(src.at[row0], rb0, sem0)
            pltpu.async_copy(src.at[row0 + 1], rb1, sem1)

            def trip(p, c):
                r0 = row0 + 3 * p
                for u in range(3):
                    nxt = r0 + u + 2
                    @pl.when(nxt < row0 + nrows)
                    def _():
                        pltpu.async_copy(src.at[nxt], rbs[(u + 2) % 3],
                                         sems[(u + 2) % 3])
                    pltpu.make_async_copy(src.at[r0 + u], rbs[u],
                                          sems[u]).wait()
                    gather_row(rbs[u])
                    pltpu.sync_copy(ob, dst.at[r0 + u])
                return c
            lax.fori_loop(0, nrows // 3, trip, 0)
            rest = nrows - (nrows // 3) * 3
            for u in range(rest):
                r = row0 + (nrows // 3) * 3 + u
                pltpu.make_async_copy(src.at[r], rbs[u], sems[u]).wait()
                gather_row(rbs[u])
                pltpu.sync_copy(ob, dst.at[r])

        phase(xT_hbm, xoT_hbm, b * D + h * (D // 2), D // 2)
        phase(xvT_hbm, xvoT_hbm, b * Dv + h * (Dv // 2), Dv // 2)

    return pl.kernel(
        body,
        out_type=(
            jax.ShapeDtypeStruct((B * D, K), jnp.float32),
            jax.ShapeDtypeStruct((B * Dv, K), jnp.float32),
        ),
        mesh=mesh,
        scratch_types=[
            pltpu.VMEM((K,), jnp.int32),
            pltpu.VMEM((N,), jnp.float32),
            pltpu.VMEM((N,), jnp.float32),
            pltpu.VMEM((N,), jnp.float32),
            pltpu.VMEM((K,), jnp.float32),
            pltpu.SemaphoreType.DMA,
            pltpu.SemaphoreType.DMA,
            pltpu.SemaphoreType.DMA,
        ],
        compiler_params=pltpu.CompilerParams(needs_layout_passes=False),
    )


def kernel(x, x_v, x_s):
    B, N, D = x.shape
    Dv = x_v.shape[2]
    K = N // 8

    xs_bits = lax.bitcast_convert_type(x_s.reshape(B * N), jnp.int32)
    gidx, xso_bits = _make_topk(B, N, K)(xs_bits)
    xs_out = lax.bitcast_convert_type(xso_bits, jnp.float32).reshape(B, K, 1)
    xT = jnp.swapaxes(x, 1, 2).reshape(B * D, N)
    xvT = jnp.swapaxes(x_v, 1, 2).reshape(B * Dv, N)
    xoT, xvoT = _make_gather(B, N, K, D, Dv)(xT, xvT, gidx)
    xo = jnp.swapaxes(xoT.reshape(B, D, K), 1, 2)
    xvo = jnp.swapaxes(xvoT.reshape(B, Dv, K), 1, 2)
    return xo, xvo, xs_out


# speedup vs baseline: 1.0295x; 1.0295x over previous
import jax
import jax.numpy as jnp
from jax import lax
from jax.experimental import pallas as pl
from jax.experimental.pallas import tpu as pltpu
from jax.experimental.pallas import tpu_sc as plsc

_BIG = jnp.int32(0x7FFFFFFF)


def _make_topk(B, N, K):
    info = plsc.get_sparse_core_info()
    NC, NS = info.num_cores, info.num_subcores
    mesh = plsc.VectorSubcoreMesh(core_axis_name="c", subcore_axis_name="s")
    NV = N // 16
    KV = K // 16

    def body(xs_hbm, gidx_hbm, xso_hbm, kd_ref, bb_ref, h16_ref, ck_ref,
             ci_ref, ak_ref, ai_ref, hist_ref, sbuf_ref):
        wid = lax.axis_index("s") * NC + lax.axis_index("c")
        lane = lax.iota(jnp.int32, 16)

        def _dedup(d, valid):
            if valid is None:
                ukey = d * 16 + lane
            else:
                ukey = jnp.where(valid, d * 16 + lane, _BIG)
            (sk,) = lax.sort([ukey], dimension=0, num_keys=1)
            sbuf_ref[pl.ds(0, 16)] = jnp.full((16,), -1, jnp.int32)
            sbuf_ref[pl.ds(1, 16)] = sk
            prev = plsc.load_gather(sbuf_ref, [lane])
            nxt = plsc.load_gather(sbuf_ref, [jnp.minimum(lane + 2, 16)])
            sd = sk >> 4
            boundary = sd != (prev >> 4)
            fpos = plsc.cummax(jnp.where(boundary, lane, 0))
            rk_sorted = lane - fpos
            lastm = (lane == 15) | ((nxt >> 4) != sd)
            if valid is not None:
                lastm = lastm & (sk != _BIG)
            return sk, sd, lastm, rk_sorted

        def _hist_add(d, valid):
            _, sd, lastm, rk_sorted = _dedup(d, valid)
            plsc.addupdate_scatter(hist_ref, [sd], rk_sorted + 1, mask=lastm)

        def _zero_hist(nb):
            def z(i, c):
                hist_ref[pl.ds(i * 16, 16)] = jnp.zeros((16,), jnp.int32)
                return c
            lax.fori_loop(0, nb // 16, z, 0)

        def _scan_hist(nb, t):
            def sb(i, carry):
                run, nf, g = carry
                h = hist_ref[pl.ds(i * 16, 16)]
                c = plsc.cumsum(h) + run
                le = c <= t
                nf = nf + jnp.sum(jnp.where(le, 1, 0).astype(jnp.int32))
                g = jnp.maximum(g, jnp.max(jnp.where(le, c, 0)))
                return jnp.max(c), nf, g
            z = jnp.int32(0)
            _, bstar, g = lax.fori_loop(0, nb // 16, sb, (z, z, z))
            return bstar, g

        @pl.when(wid < B)
        def _():
            r = wid
            pltpu.sync_copy(xs_hbm.at[pl.ds(r * N, N)], kd_ref)

            def z1(i, c):
                for u in range(8):
                    h16_ref[pl.ds(i * 128 + u * 16, 16)] = (
                        jnp.zeros((16,), jnp.int32))
                return c
            lax.fori_loop(0, 256, z1, 0)

            def p1(i, c):
                for u in range(4):
                    uu = kd_ref[pl.ds(i * 64 + u * 16, 16)]
                    kd = ~(uu ^ ((uu >> 31) & _BIG))
                    kd_ref[pl.ds(i * 64 + u * 16, 16)] = kd
                    a = lane * 2048 + ((kd >> 21) + 1024)
                    plsc.addupdate_scatter(h16_ref, [a],
                                           jnp.full((16,), 1, jnp.int32))
                return c
            lax.fori_loop(0, NV // 4, p1, 0)

            t1 = jnp.int32(K - 1)

            def s1(j, carry):
                run, nf, g = carry
                tot = h16_ref[pl.ds(j * 16, 16)]
                for l in range(1, 16):
                    tot = tot + h16_ref[pl.ds(l * 2048 + j * 16, 16)]
                c = plsc.cumsum(tot) + run
                le = c <= t1
                nf = nf + jnp.sum(jnp.where(le, 1, 0).astype(jnp.int32))
                g = jnp.maximum(g, jnp.max(jnp.where(le, c, 0)))
                return jnp.max(c), nf, g
            z = jnp.int32(0)
            _, b1, g1 = lax.fori_loop(0, 128, s1, (z, z, z))
            b1v = b1 - 1024
            t2 = t1 - g1

            def cbb(i, off):
                for u in range(4):
                    kd = kd_ref[pl.ds(i * 64 + u * 16, 16)]
                    m = (kd >> 21) == b1v
                    plsc.store_compressed(bb_ref.at[pl.ds(off, 16)], kd,
                                          mask=m)
                    off = off + jnp.sum(jnp.where(m, 1, 0).astype(jnp.int32))
                return off
            m1 = lax.fori_loop(0, NV // 4, cbb, jnp.int32(0))
            m1v = (m1 + 15) // 16

            _zero_hist(2048)

            def h2(i, c):
                kd = bb_ref[pl.ds(i * 16, 16)]
                _hist_add((kd >> 10) & 0x7FF, (i * 16 + lane) < m1)
                return c
            lax.fori_loop(0, m1v, h2, 0)
            b2, g2 = _scan_hist(2048, t2)
            t3 = t2 - g2

            def c2(i, off):
                kd = bb_ref[pl.ds(i * 16, 16)]
                m = ((i * 16 + lane) < m1) & (((kd >> 10) & 0x7FF) == b2)
                plsc.store_compressed(bb_ref.at[pl.ds(off, 16)], kd, mask=m)
                return off + jnp.sum(jnp.where(m, 1, 0).astype(jnp.int32))
            m2 = lax.fori_loop(0, m1v, c2, jnp.int32(0))
            m2v = (m2 + 15) // 16

            _zero_hist(1024)

            def h3(i, c):
                kd = bb_ref[pl.ds(i * 16, 16)]
                _hist_add(kd & 0x3FF, (i * 16 + lane) < m2)
                return c
            lax.fori_loop(0, m2v, h3, 0)
            b3, g3 = _scan_hist(1024, t3)

            T = (b1v << 21) | (b2 << 10) | b3
            need = jnp.int32(K) - (g1 + g2 + g3)

            def cb(i, carry):
                off, trun = carry
                for u in range(4):
                    kd = kd_ref[pl.ds(i * 64 + u * 16, 16)]
                    strict = kd < T
                    tie = kd == T
                    tord = plsc.cumsum(jnp.where(tie, 1, 0).astype(jnp.int32))
                    msel = strict | (tie & ((tord + trun) <= need))
                    gi = i * 64 + u * 16 + lane
                    plsc.store_compressed(ck_ref.at[pl.ds(off, 16)], kd,
                                          mask=msel)
                    plsc.store_compressed(ci_ref.at[pl.ds(off, 16)], gi,
                                          mask=msel)
                    off = off + jnp.sum(jnp.where(msel, 1, 0).astype(jnp.int32))
                    trun = trun + jnp.max(tord)
                return off, trun
            lax.fori_loop(0, NV // 4, cb, (jnp.int32(0), jnp.int32(0)))

            for p in range(4):
                src_k, src_i = (ck_ref, ci_ref) if p % 2 == 0 else (ak_ref, ai_ref)
                dst_k, dst_i = (ak_ref, ai_ref) if p % 2 == 0 else (ck_ref, ci_ref)
                sh = 8 * p

                def dig(kd, sh=sh, p=p):
                    d = (kd >> sh) & 255
                    if p == 3:
                        d = d ^ 128
                    return d

                def zr(i, c):
                    for u in range(8):
                        h16_ref[pl.ds(i * 128 + u * 16, 16)] = (
                            jnp.zeros((16,), jnp.int32))
                    return c
                lax.fori_loop(0, 128, zr, 0)

                q = KV // 4

                def hb(i, c, src_k=src_k, dig=dig):
                    for u in range(4):
                        kd = plsc.load_gather(src_k, [lane * KV + u * q + i])
                        a = u * 4096 + dig(kd) * 16 + lane
                        plsc.addupdate_scatter(h16_ref, [a],
                                               jnp.full((16,), 1, jnp.int32))
                    return c
                lax.fori_loop(0, q, hb, 0)

                def pb(d, run):
                    v0 = h16_ref[pl.ds(d * 16, 16)]
                    v1 = h16_ref[pl.ds(4096 + d * 16, 16)]
                    v2 = h16_ref[pl.ds(8192 + d * 16, 16)]
                    v3 = h16_ref[pl.ds(12288 + d * 16, 16)]
                    tot = v0 + v1 + v2 + v3
                    acc = plsc.cumsum(tot) - tot + run
                    bb_ref[pl.ds(d * 16, 16)] = acc
                    acc = acc + v0
                    bb_ref[pl.ds(4096 + d * 16, 16)] = acc
                    acc = acc + v1
                    bb_ref[pl.ds(8192 + d * 16, 16)] = acc
                    acc = acc + v2
                    bb_ref[pl.ds(12288 + d * 16, 16)] = acc
                    return run + jnp.sum(tot)
                lax.fori_loop(0, 256, pb, jnp.int32(0))

                def mb(i, c, src_k=src_k, src_i=src_i, dst_k=dst_k,
                       dst_i=dst_i, dig=dig):
                    for u in range(4):
                        kd = plsc.load_gather(src_k, [lane * KV + u * q + i])
                        ix = plsc.load_gather(src_i, [lane * KV + u * q + i])
                        a = u * 4096 + dig(kd) * 16 + lane
                        pos = plsc.load_gather(bb_ref, [a])
                        plsc.store_scatter(bb_ref, [a], pos + 1)
                        plsc.store_scatter(dst_k, [pos], kd)
                        plsc.store_scatter(dst_i, [pos], ix)
                    return c
                lax.fori_loop(0, q, mb, 0)

            def ob(i, c):
                for u in range(4):
                    kd = ck_ref[pl.ds(i * 64 + u * 16, 16)]
                    s = ~kd
                    ck_ref[pl.ds(i * 64 + u * 16, 16)] = s ^ ((s >> 31) & _BIG)
                return c
            lax.fori_loop(0, KV // 4, ob, 0)
            pltpu.sync_copy(ck_ref.at[pl.ds(0, K)], xso_hbm.at[pl.ds(r * K, K)])
            pltpu.sync_copy(ci_ref.at[pl.ds(0, K)], gidx_hbm.at[pl.ds(r * K, K)])

    return pl.kernel(
        body,
        out_type=(
            jax.ShapeDtypeStruct((B * K,), jnp.int32),
            jax.ShapeDtypeStruct((B * K,), jnp.int32),
        ),
        mesh=mesh,
        scratch_types=[
            pltpu.VMEM((N,), jnp.int32),
            pltpu.VMEM((N,), jnp.int32),
            pltpu.VMEM((16 * 2048,), jnp.int32),
            pltpu.VMEM((K + 16,), jnp.int32),
            pltpu.VMEM((K + 16,), jnp.int32),
            pltpu.VMEM((K + 16,), jnp.int32),
            pltpu.VMEM((K + 16,), jnp.int32),
            pltpu.VMEM((2048,), jnp.int32),
            pltpu.VMEM((48,), jnp.int32),
        ],
        compiler_params=pltpu.CompilerParams(needs_layout_passes=False),
    )


def _make_gather(B, N, K, D, Dv):
    info = plsc.get_sparse_core_info()
    NC, NS = info.num_cores, info.num_subcores
    mesh = plsc.VectorSubcoreMesh(core_axis_name="c", subcore_axis_name="s")
    KV = K // 16

    def body(xT_hbm, xvT_hbm, gidx_hbm, xoT_hbm, xvoT_hbm, idx_ref, rb0, rb1,
             rb2, ob, sem0, sem1, sem2):
        wid = lax.axis_index("s") * NC + lax.axis_index("c")
        b = wid // 2
        h = wid % 2

        pltpu.sync_copy(gidx_hbm.at[pl.ds(b * K, K)], idx_ref)

        def gather_row(rbuf):
            def g(j, c):
                for u in range(4):
                    iv = idx_ref[pl.ds(j * 64 + u * 16, 16)]
                    ob[pl.ds(j * 64 + u * 16, 16)] = (
                        plsc.load_gather(rbuf, [iv]))
                return c
            lax.fori_loop(0, KV // 4, g, 0)

        rbs = (rb0, rb1, rb2)
        sems = (sem0, sem1, sem2)

        def phase(src, dst, row0, nrows):
            pltpu.async_copy(src.at[row0], rb0, sem0)
            pltpu.async_copy(src.at[row0 + 1], rb1, sem1)

            def trip(p, c):
                r0 = row0 + 3 * p
                for u in range(3):
                    nxt = r0 + u + 2
                    @pl.when(nxt < row0 + nrows)
                    def _():
                        pltpu.async_copy(src.at[nxt], rbs[(u + 2) % 3],
                                         sems[(u + 2) % 3])
                    pltpu.make_async_copy(src.at[r0 + u], rbs[u],
                                          sems[u]).wait()
                    gather_row(rbs[u])
                    pltpu.sync_copy(ob, dst.at[r0 + u])
                return c
            lax.fori_loop(0, nrows // 3, trip, 0)
            rest = nrows - (nrows // 3) * 3
            for u in range(rest):
                r = row0 + (nrows // 3) * 3 + u
                pltpu.make_async_copy(src.at[r], rbs[u], sems[u]).wait()
                gather_row(rbs[u])
                pltpu.sync_copy(ob, dst.at[r])

        phase(xT_hbm, xoT_hbm, b * D + h * (D // 2), D // 2)
        phase(xvT_hbm, xvoT_hbm, b * Dv + h * (Dv // 2), Dv // 2)

    return pl.kernel(
        body,
        out_type=(
            jax.ShapeDtypeStruct((B * D, K), jnp.float32),
            jax.ShapeDtypeStruct((B * Dv, K), jnp.float32),
        ),
        mesh=mesh,
        scratch_types=[
            pltpu.VMEM((K,), jnp.int32),
            pltpu.VMEM((N,), jnp.float32),
            pltpu.VMEM((N,), jnp.float32),
            pltpu.VMEM((N,), jnp.float32),
            pltpu.VMEM((K,), jnp.float32),
            pltpu.SemaphoreType.DMA,
            pltpu.SemaphoreType.DMA,
            pltpu.SemaphoreType.DMA,
        ],
        compiler_params=pltpu.CompilerParams(needs_layout_passes=False),
    )


def kernel(x, x_v, x_s):
    B, N, D = x.shape
    Dv = x_v.shape[2]
    K = N // 8

    xs_bits = lax.bitcast_convert_type(x_s.reshape(B * N), jnp.int32)
    gidx, xso_bits = _make_topk(B, N, K)(xs_bits)
    xs_out = lax.bitcast_convert_type(xso_bits, jnp.float32).reshape(B, K, 1)
    xT = jnp.swapaxes(x, 1, 2).reshape(B * D, N)
    xvT = jnp.swapaxes(x_v, 1, 2).reshape(B * Dv, N)
    xoT, xvoT = _make_gather(B, N, K, D, Dv)(xT, xvT, gidx)
    xo = jnp.swapaxes(xoT.reshape(B, D, K), 1, 2)
    xvo = jnp.swapaxes(xvoT.reshape(B, Dv, K), 1, 2)
    return xo, xvo, xs_out


# speedup vs baseline: 1.0312x; 1.0016x over previous
import jax
import jax.numpy as jnp
from jax import lax
from jax.experimental import pallas as pl
from jax.experimental.pallas import tpu as pltpu
from jax.experimental.pallas import tpu_sc as plsc

_BIG = jnp.int32(0x7FFFFFFF)


def _make_topk(B, N, K):
    info = plsc.get_sparse_core_info()
    NC, NS = info.num_cores, info.num_subcores
    mesh = plsc.VectorSubcoreMesh(core_axis_name="c", subcore_axis_name="s")
    NV = N // 16
    KV = K // 16

    def body(xs_hbm, gidx_hbm, xso_hbm, kd_ref, bb_ref, h16_ref, ck_ref,
             ci_ref, ak_ref, ai_ref, hist_ref, sbuf_ref):
        wid = lax.axis_index("s") * NC + lax.axis_index("c")
        lane = lax.iota(jnp.int32, 16)

        def _dedup(d, valid):
            if valid is None:
                ukey = d * 16 + lane
            else:
                ukey = jnp.where(valid, d * 16 + lane, _BIG)
            (sk,) = lax.sort([ukey], dimension=0, num_keys=1)
            sbuf_ref[pl.ds(0, 16)] = jnp.full((16,), -1, jnp.int32)
            sbuf_ref[pl.ds(1, 16)] = sk
            prev = plsc.load_gather(sbuf_ref, [lane])
            nxt = plsc.load_gather(sbuf_ref, [jnp.minimum(lane + 2, 16)])
            sd = sk >> 4
            boundary = sd != (prev >> 4)
            fpos = plsc.cummax(jnp.where(boundary, lane, 0))
            rk_sorted = lane - fpos
            lastm = (lane == 15) | ((nxt >> 4) != sd)
            if valid is not None:
                lastm = lastm & (sk != _BIG)
            return sk, sd, lastm, rk_sorted

        def _hist_add(d, valid):
            _, sd, lastm, rk_sorted = _dedup(d, valid)
            plsc.addupdate_scatter(hist_ref, [sd], rk_sorted + 1, mask=lastm)

        def _zero_hist(nb):
            def z(i, c):
                hist_ref[pl.ds(i * 16, 16)] = jnp.zeros((16,), jnp.int32)
                return c
            lax.fori_loop(0, nb // 16, z, 0)

        def _scan_hist(nb, t):
            def sb(i, carry):
                run, nf, g = carry
                h = hist_ref[pl.ds(i * 16, 16)]
                c = plsc.cumsum(h) + run
                le = c <= t
                nf = nf + jnp.sum(jnp.where(le, 1, 0).astype(jnp.int32))
                g = jnp.maximum(g, jnp.max(jnp.where(le, c, 0)))
                return jnp.max(c), nf, g
            z = jnp.int32(0)
            _, bstar, g = lax.fori_loop(0, nb // 16, sb, (z, z, z))
            return bstar, g

        @pl.when(wid < B)
        def _():
            r = wid
            pltpu.sync_copy(xs_hbm.at[pl.ds(r * N, N)], kd_ref)

            def z1(i, c):
                for u in range(8):
                    h16_ref[pl.ds(i * 128 + u * 16, 16)] = (
                        jnp.zeros((16,), jnp.int32))
                return c
            lax.fori_loop(0, 256, z1, 0)

            def p1(i, c):
                for u in range(4):
                    uu = kd_ref[pl.ds(i * 64 + u * 16, 16)]
                    kd = ~(uu ^ ((uu >> 31) & _BIG))
                    kd_ref[pl.ds(i * 64 + u * 16, 16)] = kd
                    a = lane * 2048 + ((kd >> 21) + 1024)
                    plsc.addupdate_scatter(h16_ref, [a],
                                           jnp.full((16,), 1, jnp.int32))
                return c
            lax.fori_loop(0, NV // 4, p1, 0)

            t1 = jnp.int32(K - 1)

            def s1(j, carry):
                run, nf, g = carry
                tot = h16_ref[pl.ds(j * 16, 16)]
                for l in range(1, 16):
                    tot = tot + h16_ref[pl.ds(l * 2048 + j * 16, 16)]
                c = plsc.cumsum(tot) + run
                le = c <= t1
                nf = nf + jnp.sum(jnp.where(le, 1, 0).astype(jnp.int32))
                g = jnp.maximum(g, jnp.max(jnp.where(le, c, 0)))
                return jnp.max(c), nf, g
            z = jnp.int32(0)
            _, b1, g1 = lax.fori_loop(0, 128, s1, (z, z, z))
            b1v = b1 - 1024
            t2 = t1 - g1

            def cbb(i, off):
                for u in range(4):
                    kd = kd_ref[pl.ds(i * 64 + u * 16, 16)]
                    m = (kd >> 21) == b1v
                    plsc.store_compressed(bb_ref.at[pl.ds(off, 16)], kd,
                                          mask=m)
                    off = off + jnp.sum(jnp.where(m, 1, 0).astype(jnp.int32))
                return off
            m1 = lax.fori_loop(0, NV // 4, cbb, jnp.int32(0))
            m1v = (m1 + 15) // 16

            _zero_hist(2048)

            def h2(i, c):
                kd = bb_ref[pl.ds(i * 16, 16)]
                _hist_add((kd >> 10) & 0x7FF, (i * 16 + lane) < m1)
                return c
            lax.fori_loop(0, m1v, h2, 0)
            b2, g2 = _scan_hist(2048, t2)
            t3 = t2 - g2

            def c2(i, off):
                kd = bb_ref[pl.ds(i * 16, 16)]
                m = ((i * 16 + lane) < m1) & (((kd >> 10) & 0x7FF) == b2)
                plsc.store_compressed(bb_ref.at[pl.ds(off, 16)], kd, mask=m)
                return off + jnp.sum(jnp.where(m, 1, 0).astype(jnp.int32))
            m2 = lax.fori_loop(0, m1v, c2, jnp.int32(0))
            m2v = (m2 + 15) // 16

            _zero_hist(1024)

            def h3(i, c):
                kd = bb_ref[pl.ds(i * 16, 16)]
                _hist_add(kd & 0x3FF, (i * 16 + lane) < m2)
                return c
            lax.fori_loop(0, m2v, h3, 0)
            b3, g3 = _scan_hist(1024, t3)

            T = (b1v << 21) | (b2 << 10) | b3
            need = jnp.int32(K) - (g1 + g2 + g3)

            def cb(i, carry):
                off, trun = carry
                for u in range(4):
                    kd = kd_ref[pl.ds(i * 64 + u * 16, 16)]
                    strict = kd < T
                    tie = kd == T
                    tord = plsc.cumsum(jnp.where(tie, 1, 0).astype(jnp.int32))
                    msel = strict | (tie & ((tord + trun) <= need))
                    gi = i * 64 + u * 16 + lane
                    plsc.store_compressed(ck_ref.at[pl.ds(off, 16)], kd,
                                          mask=msel)
                    plsc.store_compressed(ci_ref.at[pl.ds(off, 16)], gi,
                                          mask=msel)
                    off = off + jnp.sum(jnp.where(msel, 1, 0).astype(jnp.int32))
                    trun = trun + jnp.max(tord)
                return off, trun
            lax.fori_loop(0, NV // 4, cb, (jnp.int32(0), jnp.int32(0)))

            for p in range(4):
                src_k, src_i = (ck_ref, ci_ref) if p % 2 == 0 else (ak_ref, ai_ref)
                dst_k, dst_i = (ak_ref, ai_ref) if p % 2 == 0 else (ck_ref, ci_ref)
                sh = 8 * p

                def dig(kd, sh=sh, p=p):
                    d = (kd >> sh) & 255
                    if p == 3:
                        d = d ^ 128
                    return d

                def zr(i, c):
                    for u in range(8):
                        h16_ref[pl.ds(i * 128 + u * 16, 16)] = (
                            jnp.zeros((16,), jnp.int32))
                    return c
                lax.fori_loop(0, 128, zr, 0)

                q = KV // 4

                def hb(i, c, src_k=src_k, dig=dig):
                    for u in range(4):
                        kd = plsc.load_gather(src_k, [lane * KV + u * q + i])
                        a = u * 4096 + dig(kd) * 16 + lane
                        plsc.addupdate_scatter(h16_ref, [a],
                                               jnp.full((16,), 1, jnp.int32))
                    return c
                lax.fori_loop(0, q, hb, 0)

                def pb(d, run):
                    v0 = h16_ref[pl.ds(d * 16, 16)]
                    v1 = h16_ref[pl.ds(4096 + d * 16, 16)]
                    v2 = h16_ref[pl.ds(8192 + d * 16, 16)]
                    v3 = h16_ref[pl.ds(12288 + d * 16, 16)]
                    tot = v0 + v1 + v2 + v3
                    acc = plsc.cumsum(tot) - tot + run
                    bb_ref[pl.ds(d * 16, 16)] = acc
                    acc = acc + v0
                    bb_ref[pl.ds(4096 + d * 16, 16)] = acc
                    acc = acc + v1
                    bb_ref[pl.ds(8192 + d * 16, 16)] = acc
                    acc = acc + v2
                    bb_ref[pl.ds(12288 + d * 16, 16)] = acc
                    return run + jnp.sum(tot)
                lax.fori_loop(0, 256, pb, jnp.int32(0))

                def mb(i, c, src_k=src_k, src_i=src_i, dst_k=dst_k,
                       dst_i=dst_i, dig=dig):
                    for u in range(4):
                        kd = plsc.load_gather(src_k, [lane * KV + u * q + i])
                        ix = plsc.load_gather(src_i, [lane * KV + u * q + i])
                        a = u * 4096 + dig(kd) * 16 + lane
                        pos = plsc.load_gather(bb_ref, [a])
                        plsc.store_scatter(bb_ref, [a], pos + 1)
                        plsc.store_scatter(dst_k, [pos], kd)
                        plsc.store_scatter(dst_i, [pos], ix)
                    return c
                lax.fori_loop(0, q, mb, 0)

            def ob(i, c):
                for u in range(4):
                    kd = ck_ref[pl.ds(i * 64 + u * 16, 16)]
                    s = ~kd
                    ck_ref[pl.ds(i * 64 + u * 16, 16)] = s ^ ((s >> 31) & _BIG)
                return c
            lax.fori_loop(0, KV // 4, ob, 0)
            pltpu.sync_copy(ck_ref.at[pl.ds(0, K)], xso_hbm.at[pl.ds(r * K, K)])
            pltpu.sync_copy(ci_ref.at[pl.ds(0, K)], gidx_hbm.at[pl.ds(r * K, K)])

    return pl.kernel(
        body,
        out_type=(
            jax.ShapeDtypeStruct((B * K,), jnp.int32),
            jax.ShapeDtypeStruct((B * K,), jnp.int32),
        ),
        mesh=mesh,
        scratch_types=[
            pltpu.VMEM((N,), jnp.int32),
            pltpu.VMEM((N,), jnp.int32),
            pltpu.VMEM((16 * 2048,), jnp.int32),
            pltpu.VMEM((K + 16,), jnp.int32),
            pltpu.VMEM((K + 16,), jnp.int32),
            pltpu.VMEM((K + 16,), jnp.int32),
            pltpu.VMEM((K + 16,), jnp.int32),
            pltpu.VMEM((2048,), jnp.int32),
            pltpu.VMEM((48,), jnp.int32),
        ],
        compiler_params=pltpu.CompilerParams(needs_layout_passes=False),
    )


def _make_gather(B, N, K, D, Dv):
    info = plsc.get_sparse_core_info()
    NC, NS = info.num_cores, info.num_subcores
    mesh = plsc.VectorSubcoreMesh(core_axis_name="c", subcore_axis_name="s")
    KV = K // 16

    def body(xT_hbm, xvT_hbm, gidx_hbm, xoT_hbm, xvoT_hbm, idx_ref, rb0, rb1,
             rb2, ob0, ob1, ob2, sem0, sem1, sem2, os0, os1, os2):
        wid = lax.axis_index("s") * NC + lax.axis_index("c")
        b = wid // 2
        h = wid % 2

        pltpu.sync_copy(gidx_hbm.at[pl.ds(b * K, K)], idx_ref)

        def gather_row(rbuf, obuf):
            def g(j, c):
                for u in range(4):
                    iv = idx_ref[pl.ds(j * 64 + u * 16, 16)]
                    obuf[pl.ds(j * 64 + u * 16, 16)] = (
                        plsc.load_gather(rbuf, [iv]))
                return c
            lax.fori_loop(0, KV // 4, g, 0)

        rbs = (rb0, rb1, rb2)
        obs = (ob0, ob1, ob2)
        sems = (sem0, sem1, sem2)
        osems = (os0, os1, os2)

        def phase(src, dst, row0, nrows):
            pltpu.async_copy(src.at[row0], rb0, sem0)
            pltpu.async_copy(src.at[row0 + 1], rb1, sem1)

            def trip(p, c):
                r0 = row0 + 3 * p
                for u in range(3):
                    nxt = r0 + u + 2
                    @pl.when(nxt < row0 + nrows)
                    def _():
                        pltpu.async_copy(src.at[nxt], rbs[(u + 2) % 3],
                                         sems[(u + 2) % 3])
                    pltpu.make_async_copy(src.at[r0 + u], rbs[u],
                                          sems[u]).wait()

                    gather_row(rbs[u], obs[u])
                    pltpu.sync_copy(obs[u], dst.at[r0 + u])
                return c
            ntrip = nrows // 3
            lax.fori_loop(0, ntrip, trip, 0)
            rest = nrows - ntrip * 3
            for u in range(rest):
                r = row0 + ntrip * 3 + u
                pltpu.make_async_copy(src.at[r], rbs[u], sems[u]).wait()
                gather_row(rbs[u], obs[u])
                pltpu.sync_copy(obs[u], dst.at[r])

        phase(xT_hbm, xoT_hbm, b * D + h * (D // 2), D // 2)
        phase(xvT_hbm, xvoT_hbm, b * Dv + h * (Dv // 2), Dv // 2)

    return pl.kernel(
        body,
        out_type=(
            jax.ShapeDtypeStruct((B * D, K), jnp.float32),
            jax.ShapeDtypeStruct((B * Dv, K), jnp.float32),
        ),
        mesh=mesh,
        scratch_types=[
            pltpu.VMEM((K,), jnp.int32),
            pltpu.VMEM((N,), jnp.float32),
            pltpu.VMEM((N,), jnp.float32),
            pltpu.VMEM((N,), jnp.float32),
            pltpu.VMEM((K,), jnp.float32),
            pltpu.VMEM((K,), jnp.float32),
            pltpu.VMEM((K,), jnp.float32),
            pltpu.SemaphoreType.DMA,
            pltpu.SemaphoreType.DMA,
            pltpu.SemaphoreType.DMA,
            pltpu.SemaphoreType.DMA,
            pltpu.SemaphoreType.DMA,
            pltpu.SemaphoreType.DMA,
        ],
        compiler_params=pltpu.CompilerParams(needs_layout_passes=False),
    )


def kernel(x, x_v, x_s):
    B, N, D = x.shape
    Dv = x_v.shape[2]
    K = N // 8

    xs_bits = lax.bitcast_convert_type(x_s.reshape(B * N), jnp.int32)
    gidx, xso_bits = _make_topk(B, N, K)(xs_bits)
    xs_out = lax.bitcast_convert_type(xso_bits, jnp.float32).reshape(B, K, 1)
    xT = jnp.swapaxes(x, 1, 2).reshape(B * D, N)
    xvT = jnp.swapaxes(x_v, 1, 2).reshape(B * Dv, N)
    xoT, xvoT = _make_gather(B, N, K, D, Dv)(xT, xvT, gidx)
    xo = jnp.swapaxes(xoT.reshape(B, D, K), 1, 2)
    xvo = jnp.swapaxes(xvoT.reshape(B, Dv, K), 1, 2)
    return xo, xvo, xs_out
